# baseline (speedup 1.0000x reference)
import functools

import jax
import jax.numpy as jnp
from jax import lax
from jax.experimental import pallas as pl
from jax.experimental.pallas import tpu as pltpu
from jax.experimental.pallas import tpu_sc as plsc

BATCH = 16384
D = 16
OUT_W = 3 * D

_info = plsc.get_sparse_core_info()
NC, NS = _info.num_cores, _info.num_subcores
NW = NC * NS
BPW = BATCH // NW

NBUF = 16
NGRP = BPW // NBUF

_mesh = plsc.VectorSubcoreMesh(core_axis_name="c", subcore_axis_name="s")


@functools.partial(
    pl.kernel,
    mesh=_mesh,
    out_type=jax.ShapeDtypeStruct((OUT_W, BATCH), jnp.float32),
    compiler_params=pltpu.CompilerParams(needs_layout_passes=False),
    scratch_types=[
        pltpu.VMEM((BPW,), jnp.int32),
        pltpu.VMEM((BPW,), jnp.int32),
        pltpu.VMEM((BPW,), jnp.int32),
        pltpu.VMEM((NBUF, D, 128), jnp.float32),
        pltpu.VMEM((NBUF, 8, 128), jnp.int32),
        pltpu.VMEM((OUT_W, BPW), jnp.float32),
        pltpu.VMEM_SHARED((8, 100000), jnp.int32),
        [pltpu.SemaphoreType.DMA] * NBUF,
        pltpu.SemaphoreType.DMA,
    ],
)
def _gather3(uid, iid, tid, ut, it, tbf, out, uix, iix, tix, tile_v, tile_bf, cat_v, spm, sems, fsem):
    sid = lax.axis_index("s")
    wid = sid * NC + lax.axis_index("c")
    base = wid * BPW

    @pl.when(sid == 0)
    def _():
        pltpu.async_copy(tbf, spm, fsem).wait()

    for ids, ivec in ((uid, uix), (iid, iix), (tid, tix)):
        pltpu.sync_copy(ids.at[pl.ds(base, BPW)], ivec)
    d_iota = lax.iota(jnp.int32, 16)
    plsc.subcore_barrier()

    for t, (tab, ivec, is_bf) in enumerate(
        ((ut, uix, False), (it, iix, False), (spm, tix, True))
    ):
        tiles = tile_bf if is_bf else tile_v

        def fire(col, s, tab=tab, tiles=tiles):
            pltpu.async_copy(
                tab.at[:, pl.ds(pl.multiple_of(col, 128), 128)],
                tiles.at[s],
                sems[s],
            )

        def fire_group(g, ivec=ivec, fire=fire):
            vg = ivec[pl.ds(g * NBUF, NBUF)]
            cols = (vg >> 7) * 128
            for s in range(NBUF):
                fire(cols[s], s)

        fire_group(0)

        def ring_body(g, tab=tab, t=t, ivec=ivec, tiles=tiles, is_bf=is_bf, fire=fire):
            vg = ivec[pl.ds(g * NBUF, NBUF)]
            lanes = vg & 127

            def extract(s):
                j = g * NBUF + s
                jv = jnp.broadcast_to(jnp.int32(0) + j, (16,))
                if not is_bf:
                    lane = jnp.broadcast_to(lanes[s], (16,))
                    row = plsc.load_gather(tiles.at[s], [d_iota, lane])
                else:
                    lane = jnp.broadcast_to(lanes[s], (16,))
                    w = plsc.load_gather(tiles.at[s], [d_iota >> 1, lane])
                    pair = plsc.bitcast(w, jnp.bfloat16)
                    lo, hi = plsc.unpack(pair, format=plsc.PackFormat.INTERLEAVED)
                    row = jnp.where((d_iota & 1) == 1, hi, lo)
                plsc.store_scatter(cat_v, [t * D + d_iota, jv], row)

            def wait_slot(s, tiles=tiles):
                pltpu.make_async_copy(
                    tab.at[:, pl.ds(0, 128)],
                    tiles.at[s],
                    sems[s],
                ).wait()

            @pl.when(g + 1 < NGRP)
            def _():
                vn = ivec[pl.ds((g + 1) * NBUF, NBUF)]
                cols = (vn >> 7) * 128
                for s in range(NBUF):
                    wait_slot(s)
                    extract(s)
                    fire(cols[s], s)

            @pl.when(g + 1 >= NGRP)
            def _():
                for s in range(NBUF):
                    wait_slot(s)
                    extract(s)

        pl.loop(0, NGRP)(ring_body)

    pltpu.sync_copy(
        cat_v, out.at[pl.ds(0, OUT_W), pl.ds(base, BPW)]
    )


def kernel(user_id, item_id, tag_id, user_table, item_table, tag_table):
    out_t = _gather3(
        user_id, item_id, tag_id,
        user_table.T, item_table.T,
        lax.bitcast_convert_type(
            tag_table.astype(jnp.bfloat16).reshape(100000, 8, 2), jnp.int32
        ).T,
    )
    return out_t.T
